# baseline (speedup 1.0000x reference)
import jax
import jax.numpy as jnp
from jax import lax
from jax.experimental import pallas as pl
from jax.experimental.pallas import tpu as pltpu
from jax.experimental.pallas import tpu_sc as plsc

NUM_EMBEDDINGS = 1000000
EMBED_DIM = 64
EPS = 1e-5

NC = 2
NS = 16
NW = NC * NS

B = 4096 * 200
PER_W = B // NW
CHUNK = 512
N_CHUNKS = PER_W // CHUNK
IDX_ROWS = CHUNK // 128

BLK_B = 32
RB = BLK_B * 200 // 2


def _gather_body(x_hbm, table_hbm, out_hbm, idx_v, rows_v, gsem0, gsem1):
    wid = lax.axis_index("s") * NC + lax.axis_index("c")
    idx_row0 = wid * (PER_W // 128)
    out_row0 = wid * PER_W
    gsems = (gsem0, gsem1)

    def load_idx(ci, b):
        pltpu.sync_copy(
            x_hbm.at[pl.ds(idx_row0 + ci * IDX_ROWS, IDX_ROWS)], idx_v.at[b])

    def fire(b):
        for j in range(IDX_ROWS):
            pltpu.async_copy(table_hbm.at[idx_v.at[b, j]],
                             rows_v.at[b, pl.ds(j * 128, 128)], gsems[b])

    def wait_gathers(b):
        for j in range(IDX_ROWS):
            pltpu.make_async_copy(table_hbm.at[idx_v.at[b, j]],
                                  rows_v.at[b, pl.ds(j * 128, 128)],
                                  gsems[b]).wait()

    def copy_out(ci, b):
        pltpu.sync_copy(rows_v.at[b],
                        out_hbm.at[pl.ds(out_row0 + ci * CHUNK, CHUNK)])

    def step(ci, b):
        nb = 1 - b
        load_idx(ci + 1, nb)
        fire(nb)
        wait_gathers(b)
        copy_out(ci, b)

    load_idx(0, 0)
    fire(0)

    def pair_body(k, carry):
        step(2 * k, 0)
        step(2 * k + 1, 1)
        return carry

    lax.fori_loop(0, N_CHUNKS // 2 - 1, pair_body, 0)
    step(N_CHUNKS - 2, 0)
    wait_gathers(1)
    copy_out(N_CHUNKS - 1, 1)


def _sc_gather(xf, table):
    mesh = plsc.VectorSubcoreMesh(core_axis_name="c", subcore_axis_name="s",
                                  num_cores=NC, num_subcores=NS)
    return pl.kernel(
        _gather_body,
        out_type=jax.ShapeDtypeStruct((B, EMBED_DIM), jnp.float32),
        mesh=mesh,
        compiler_params=pltpu.CompilerParams(needs_layout_passes=False,
                                             use_tc_tiling_on_sc=False),
        scratch_types=[
            pltpu.VMEM((2, IDX_ROWS, 128), jnp.int32),
            pltpu.VMEM((2, CHUNK, EMBED_DIM), jnp.float32),
            pltpu.SemaphoreType.DMA,
            pltpu.SemaphoreType.DMA,
        ],
    )(xf, table)


def _ln_body(g_ref, gamma_ref, beta_ref, out_ref):
    x = g_ref[...]
    g = gamma_ref[0, :]
    b = beta_ref[0, :]
    gb = jnp.concatenate([g, g])
    bb = jnp.concatenate([b, b])
    lane = lax.broadcasted_iota(jnp.int32, (128, 2), 0)
    half = lax.broadcasted_iota(jnp.int32, (128, 2), 1)
    sel = (lane // EMBED_DIM == half).astype(jnp.float32)
    lane_t = lax.broadcasted_iota(jnp.int32, (2, 128), 1)
    half_t = lax.broadcasted_iota(jnp.int32, (2, 128), 0)
    sel_t = (lane_t // EMBED_DIM == half_t).astype(jnp.float32)

    def mm(a, c):
        return lax.dot_general(a, c, (((1,), (0,)), ((), ())),
                               precision=lax.Precision.HIGHEST,
                               preferred_element_type=jnp.float32)

    s = mm(x, sel)
    q = mm(x * x, sel)
    mean = s * (1.0 / EMBED_DIM)
    var = q * (1.0 / EMBED_DIM) - mean * mean
    rstd = lax.rsqrt(var + EPS)
    meanb = mm(mean, sel_t)
    rstdb = mm(rstd, sel_t)
    o = (x - meanb) * rstdb * gb + bb
    out_ref[...] = o.reshape(BLK_B, 100, 128)


def _tc_layernorm(g2, gamma2, beta2):
    return pl.pallas_call(
        _ln_body,
        grid=(4096 // BLK_B,),
        in_specs=[
            pl.BlockSpec((RB, 128), lambda i: (i, 0)),
            pl.BlockSpec((1, EMBED_DIM), lambda i: (0, 0)),
            pl.BlockSpec((1, EMBED_DIM), lambda i: (0, 0)),
        ],
        out_specs=pl.BlockSpec((BLK_B, 100, 128), lambda i: (i, 0, 0)),
        out_shape=jax.ShapeDtypeStruct((4096, 100, 128), jnp.float32),
    )(g2, gamma2, beta2)


@jax.jit
def kernel(x, table, gamma, beta):
    xf = x.astype(jnp.int32).reshape(B // 128, 128)
    g2 = _sc_gather(xf, table).reshape(B // 2, 128)
    out = _tc_layernorm(g2, gamma.reshape(1, EMBED_DIM),
                        beta.reshape(1, EMBED_DIM))
    return out.reshape(4096, 200, EMBED_DIM)


# speedup vs baseline: 1.2312x; 1.2312x over previous
import jax
import jax.numpy as jnp
from jax import lax
from jax.experimental import pallas as pl
from jax.experimental.pallas import tpu as pltpu
from jax.experimental.pallas import tpu_sc as plsc

NUM_EMBEDDINGS = 1000000
EMBED_DIM = 64
EPS = 1e-5

NC = 2
NS = 16
NW = NC * NS

B = 4096 * 200
S = 4
BATCH_S = 4096 // S
BS = B // S
PER_W = BS // NW
CHUNK = 640
N_CHUNKS = PER_W // CHUNK
IDX_ROWS = CHUNK // 128

BLK_B = 32
RB = BLK_B * 200 // 2


def _gather_body(x_hbm, table_hbm, out_hbm, idx_v, rows_v, gsem0, gsem1):
    wid = lax.axis_index("s") * NC + lax.axis_index("c")
    idx_row0 = wid * (PER_W // 128)
    out_row0 = wid * PER_W
    gsems = (gsem0, gsem1)

    def load_idx(ci, b):
        pltpu.sync_copy(
            x_hbm.at[pl.ds(idx_row0 + ci * IDX_ROWS, IDX_ROWS)], idx_v.at[b])

    def fire(b):
        for j in range(IDX_ROWS):
            pltpu.async_copy(table_hbm.at[idx_v.at[b, j]],
                             rows_v.at[b, pl.ds(j * 128, 128)], gsems[b])

    def wait_gathers(b):
        for j in range(IDX_ROWS):
            pltpu.make_async_copy(table_hbm.at[idx_v.at[b, j]],
                                  rows_v.at[b, pl.ds(j * 128, 128)],
                                  gsems[b]).wait()

    def copy_out(ci, b):
        pltpu.sync_copy(rows_v.at[b],
                        out_hbm.at[pl.ds(out_row0 + ci * CHUNK, CHUNK)])

    def step(ci, b):
        nb = 1 - b
        load_idx(ci + 1, nb)
        fire(nb)
        wait_gathers(b)
        copy_out(ci, b)

    load_idx(0, 0)
    fire(0)

    def pair_body(k, carry):
        step(2 * k, 0)
        step(2 * k + 1, 1)
        return carry

    lax.fori_loop(0, N_CHUNKS // 2 - 1, pair_body, 0)
    step(N_CHUNKS - 2, 0)
    wait_gathers(1)
    copy_out(N_CHUNKS - 1, 1)


def _sc_gather(xf, table):
    mesh = plsc.VectorSubcoreMesh(core_axis_name="c", subcore_axis_name="s",
                                  num_cores=NC, num_subcores=NS)
    return pl.kernel(
        _gather_body,
        out_type=jax.ShapeDtypeStruct((BS, EMBED_DIM), jnp.float32),
        mesh=mesh,
        compiler_params=pltpu.CompilerParams(needs_layout_passes=False,
                                             use_tc_tiling_on_sc=False),
        scratch_types=[
            pltpu.VMEM((2, IDX_ROWS, 128), jnp.int32),
            pltpu.VMEM((2, CHUNK, EMBED_DIM), jnp.float32),
            pltpu.SemaphoreType.DMA,
            pltpu.SemaphoreType.DMA,
        ],
    )(xf, table)


def _ln_body(g_ref, gamma_ref, beta_ref, out_ref):
    xg = g_ref[...]
    g = gamma_ref[0, :]
    b = beta_ref[0, :]

    def norm(x):
        mean = jnp.mean(x, axis=1, keepdims=True)
        xc = x - mean
        var = jnp.mean(xc * xc, axis=1, keepdims=True)
        return xc * lax.rsqrt(var + EPS) * g + b

    o = jnp.concatenate(
        [norm(xg[:, :EMBED_DIM]), norm(xg[:, EMBED_DIM:])], axis=1)
    out_ref[...] = o.reshape(BLK_B, 100, 128)


def _tc_layernorm(g2, gamma2, beta2, nb):
    return pl.pallas_call(
        _ln_body,
        grid=(nb // BLK_B,),
        in_specs=[
            pl.BlockSpec((RB, 128), lambda i: (i, 0)),
            pl.BlockSpec((1, EMBED_DIM), lambda i: (0, 0)),
            pl.BlockSpec((1, EMBED_DIM), lambda i: (0, 0)),
        ],
        out_specs=pl.BlockSpec((BLK_B, 100, 128), lambda i: (i, 0, 0)),
        out_shape=jax.ShapeDtypeStruct((nb, 100, 128), jnp.float32),
    )(g2, gamma2, beta2)


@jax.jit
def kernel(x, table, gamma, beta):
    xi = x.astype(jnp.int32)
    g1 = gamma.reshape(1, EMBED_DIM)
    b1 = beta.reshape(1, EMBED_DIM)
    parts = []
    for s in range(S):
        xs = xi[s * BATCH_S:(s + 1) * BATCH_S].reshape(BS // 128, 128)
        g2 = _sc_gather(xs, table).reshape(BS // 2, 128)
        parts.append(_tc_layernorm(g2, g1, b1, BATCH_S))
    out = jnp.concatenate(parts, axis=0)
    return out.reshape(4096, 200, EMBED_DIM)


# speedup vs baseline: 1.3647x; 1.1085x over previous
import jax
import jax.numpy as jnp
from jax import lax
from jax.experimental import pallas as pl
from jax.experimental.pallas import tpu as pltpu
from jax.experimental.pallas import tpu_sc as plsc

NUM_EMBEDDINGS = 1000000
EMBED_DIM = 64
EPS = 1e-5

NC = 2
NS = 16
NW = NC * NS

B = 4096 * 200
S = 4
BATCH_S = 4096 // S
BS = B // S
PER_W = BS // NW
CHUNK = 640
N_CHUNKS = PER_W // CHUNK
IDX_ROWS = CHUNK // 128

BLK_B = 32
RB = BLK_B * 200 // 2


def _gather_body(x_hbm, table_hbm, out_hbm, idx_v, rows_v, gsem0, gsem1):
    wid = lax.axis_index("s") * NC + lax.axis_index("c")
    idx_row0 = wid * (PER_W // 128)
    out_row0 = wid * PER_W
    gsems = (gsem0, gsem1)

    def load_idx(ci, b):
        pltpu.sync_copy(
            x_hbm.at[pl.ds(idx_row0 + ci * IDX_ROWS, IDX_ROWS)], idx_v.at[b])

    def fire(b):
        for j in range(IDX_ROWS):
            pltpu.async_copy(table_hbm.at[idx_v.at[b, j]],
                             rows_v.at[b, pl.ds(j * 128, 128)], gsems[b])

    def wait_gathers(b):
        for j in range(IDX_ROWS):
            pltpu.make_async_copy(table_hbm.at[idx_v.at[b, j]],
                                  rows_v.at[b, pl.ds(j * 128, 128)],
                                  gsems[b]).wait()

    def copy_out(ci, b):
        pltpu.sync_copy(rows_v.at[b],
                        out_hbm.at[pl.ds(out_row0 + ci * CHUNK, CHUNK)])

    def step(ci, b):
        nb = 1 - b
        load_idx(ci + 1, nb)
        fire(nb)
        wait_gathers(b)
        copy_out(ci, b)

    load_idx(0, 0)
    fire(0)

    def pair_body(k, carry):
        step(2 * k, 0)
        step(2 * k + 1, 1)
        return carry

    lax.fori_loop(0, N_CHUNKS // 2 - 1, pair_body, 0)
    step(N_CHUNKS - 2, 0)
    wait_gathers(1)
    copy_out(N_CHUNKS - 1, 1)


def _sc_gather(xf, table):
    mesh = plsc.VectorSubcoreMesh(core_axis_name="c", subcore_axis_name="s",
                                  num_cores=NC, num_subcores=NS)
    return pl.kernel(
        _gather_body,
        out_type=jax.ShapeDtypeStruct((BS, EMBED_DIM), jnp.float32),
        mesh=mesh,
        compiler_params=pltpu.CompilerParams(needs_layout_passes=False,
                                             use_tc_tiling_on_sc=False),
        scratch_types=[
            pltpu.VMEM((2, IDX_ROWS, 128), jnp.int32),
            pltpu.VMEM((2, CHUNK, EMBED_DIM), jnp.float32),
            pltpu.SemaphoreType.DMA,
            pltpu.SemaphoreType.DMA,
        ],
    )(xf, table)


def _ln_body(g_ref, gamma_ref, beta_ref, out_ref):
    x = g_ref[...]
    g = gamma_ref[0, :]
    b = beta_ref[0, :]
    gb = jnp.concatenate([g, g])
    bb = jnp.concatenate([b, b])
    lane = lax.broadcasted_iota(jnp.int32, (128, 2), 0)
    half = lax.broadcasted_iota(jnp.int32, (128, 2), 1)
    sel = (lane // EMBED_DIM == half).astype(jnp.float32)
    lane_t = lax.broadcasted_iota(jnp.int32, (2, 128), 1)
    half_t = lax.broadcasted_iota(jnp.int32, (2, 128), 0)
    sel_t = (lane_t // EMBED_DIM == half_t).astype(jnp.float32)

    def mm(a, c):
        return lax.dot_general(a, c, (((1,), (0,)), ((), ())),
                               preferred_element_type=jnp.float32)

    s = mm(x, sel)
    q = mm(x * x, sel)
    mean = s * (1.0 / EMBED_DIM)
    var = q * (1.0 / EMBED_DIM) - mean * mean
    rstd = lax.rsqrt(var + EPS)
    meanb = mm(mean, sel_t)
    rstdb = mm(rstd, sel_t)
    o = (x - meanb) * rstdb * gb + bb
    out_ref[...] = o.reshape(BLK_B, 100, 128)


def _tc_layernorm(g2, gamma2, beta2, nb):
    return pl.pallas_call(
        _ln_body,
        grid=(nb // BLK_B,),
        in_specs=[
            pl.BlockSpec((RB, 128), lambda i: (i, 0)),
            pl.BlockSpec((1, EMBED_DIM), lambda i: (0, 0)),
            pl.BlockSpec((1, EMBED_DIM), lambda i: (0, 0)),
        ],
        out_specs=pl.BlockSpec((BLK_B, 100, 128), lambda i: (i, 0, 0)),
        out_shape=jax.ShapeDtypeStruct((nb, 100, 128), jnp.float32),
    )(g2, gamma2, beta2)


@jax.jit
def kernel(x, table, gamma, beta):
    xi = x.astype(jnp.int32)
    g1 = gamma.reshape(1, EMBED_DIM)
    b1 = beta.reshape(1, EMBED_DIM)
    out = jnp.empty((4096, 100, 128), jnp.float32)
    for s in range(S):
        xs = xi[s * BATCH_S:(s + 1) * BATCH_S].reshape(BS // 128, 128)
        g2 = _sc_gather(xs, table).reshape(BS // 2, 128)
        part = _tc_layernorm(g2, g1, b1, BATCH_S)
        out = lax.dynamic_update_slice(out, part, (s * BATCH_S, 0, 0))
    return out.reshape(4096, 200, EMBED_DIM)


# speedup vs baseline: 1.5303x; 1.1214x over previous
import jax
import jax.numpy as jnp
from jax import lax
from jax.experimental import pallas as pl
from jax.experimental.pallas import tpu as pltpu
from jax.experimental.pallas import tpu_sc as plsc

NUM_EMBEDDINGS = 1000000
EMBED_DIM = 64
EPS = 1e-5

NC = 2
NS = 16
NW = NC * NS

B = 4096 * 200
S = 4
BATCH_S = 4096 // S
BS = B // S
PER_W = BS // NW
CHUNK = 640
N_CHUNKS = PER_W // CHUNK
IDX_ROWS = CHUNK // 128

BLK_B = 32
RB = BLK_B * 200 // 2


def _gather_body(x_hbm, table_hbm, out_hbm, idx_v, rows_v, gsem0, gsem1):
    wid = lax.axis_index("s") * NC + lax.axis_index("c")
    idx_row0 = wid * (PER_W // 128)
    out_row0 = wid * PER_W
    gsems = (gsem0, gsem1)

    def load_idx(ci, b):
        pltpu.sync_copy(
            x_hbm.at[pl.ds(idx_row0 + ci * IDX_ROWS, IDX_ROWS)], idx_v.at[b])

    def fire(b):
        for j in range(IDX_ROWS):
            pltpu.async_copy(table_hbm.at[idx_v.at[b, j]],
                             rows_v.at[b, pl.ds(j * 128, 128)], gsems[b])

    def wait_gathers(b):
        for j in range(IDX_ROWS):
            pltpu.make_async_copy(table_hbm.at[idx_v.at[b, j]],
                                  rows_v.at[b, pl.ds(j * 128, 128)],
                                  gsems[b]).wait()

    def copy_out(ci, b):
        pltpu.sync_copy(rows_v.at[b],
                        out_hbm.at[pl.ds(out_row0 + ci * CHUNK, CHUNK)])

    def step(ci, b):
        nb = 1 - b
        load_idx(ci + 1, nb)
        fire(nb)
        wait_gathers(b)
        copy_out(ci, b)

    load_idx(0, 0)
    fire(0)

    def pair_body(k, carry):
        step(2 * k, 0)
        step(2 * k + 1, 1)
        return carry

    lax.fori_loop(0, N_CHUNKS // 2 - 1, pair_body, 0)
    step(N_CHUNKS - 2, 0)
    wait_gathers(1)
    copy_out(N_CHUNKS - 1, 1)


def _sc_gather(xf, table):
    mesh = plsc.VectorSubcoreMesh(core_axis_name="c", subcore_axis_name="s",
                                  num_cores=NC, num_subcores=NS)
    return pl.kernel(
        _gather_body,
        out_type=jax.ShapeDtypeStruct((BS, EMBED_DIM), jnp.float32),
        mesh=mesh,
        compiler_params=pltpu.CompilerParams(needs_layout_passes=False,
                                             use_tc_tiling_on_sc=False),
        scratch_types=[
            pltpu.VMEM((2, IDX_ROWS, 128), jnp.int32),
            pltpu.VMEM((2, CHUNK, EMBED_DIM), jnp.float32),
            pltpu.SemaphoreType.DMA,
            pltpu.SemaphoreType.DMA,
        ],
    )(xf, table)


def _ln_body(g_ref, gamma_ref, beta_ref, out_ref):
    x = g_ref[...]
    g = gamma_ref[0, :]
    b = beta_ref[0, :]
    gb = jnp.concatenate([g, g])
    bb = jnp.concatenate([b, b])
    lane = lax.broadcasted_iota(jnp.int32, (128, 2), 0)
    half = lax.broadcasted_iota(jnp.int32, (128, 2), 1)
    sel = (lane // EMBED_DIM == half).astype(jnp.float32)
    lane_t = lax.broadcasted_iota(jnp.int32, (2, 128), 1)
    half_t = lax.broadcasted_iota(jnp.int32, (2, 128), 0)
    sel_t = (lane_t // EMBED_DIM == half_t).astype(jnp.float32)

    def mm(a, c):
        return lax.dot_general(a, c, (((1,), (0,)), ((), ())),
                               preferred_element_type=jnp.float32)

    s = mm(x, sel)
    q = mm(x * x, sel)
    mean = s * (1.0 / EMBED_DIM)
    var = q * (1.0 / EMBED_DIM) - mean * mean
    rstd = lax.rsqrt(var + EPS)
    meanb = mm(mean, sel_t)
    rstdb = mm(rstd, sel_t)
    o = (x - meanb) * rstdb * gb + bb
    out_ref[...] = o.reshape(BLK_B, 100, 128)


def _tc_layernorm(g2, gamma2, beta2, s, prev):
    blk0 = s * (BATCH_S // BLK_B)
    args = [g2, gamma2, beta2]
    in_specs = [
        pl.BlockSpec((RB, 128), lambda i: (i, 0)),
        pl.BlockSpec((1, EMBED_DIM), lambda i: (0, 0)),
        pl.BlockSpec((1, EMBED_DIM), lambda i: (0, 0)),
    ]
    aliases = {}
    if prev is not None:
        args.append(prev)
        in_specs.append(pl.BlockSpec(memory_space=pl.ANY))
        aliases = {3: 0}

    def body(*refs):
        _ln_body(refs[0], refs[1], refs[2], refs[-1])

    return pl.pallas_call(
        body,
        grid=(BATCH_S // BLK_B,),
        in_specs=in_specs,
        out_specs=pl.BlockSpec((BLK_B, 100, 128),
                               lambda i: (blk0 + i, 0, 0)),
        out_shape=jax.ShapeDtypeStruct((4096, 100, 128), jnp.float32),
        input_output_aliases=aliases,
    )(*args)


@jax.jit
def kernel(x, table, gamma, beta):
    xi = x.astype(jnp.int32)
    g1 = gamma.reshape(1, EMBED_DIM)
    b1 = beta.reshape(1, EMBED_DIM)
    out = None
    for s in range(S):
        xs = xi[s * BATCH_S:(s + 1) * BATCH_S].reshape(BS // 128, 128)
        g2 = _sc_gather(xs, table).reshape(BS // 2, 128)
        out = _tc_layernorm(g2, g1, b1, s, out)
    return out.reshape(4096, 200, EMBED_DIM)


# speedup vs baseline: 1.5625x; 1.0210x over previous
import jax
import jax.numpy as jnp
from jax import lax
from jax.experimental import pallas as pl
from jax.experimental.pallas import tpu as pltpu
from jax.experimental.pallas import tpu_sc as plsc

NUM_EMBEDDINGS = 1000000
EMBED_DIM = 64
EPS = 1e-5

NC = 2
NS = 16
NW = NC * NS

B = 4096 * 200
S = 4
BATCH_S = 4096 // S
BS = B // S
PER_W = BS // NW
CHUNK = 640
N_CHUNKS = PER_W // CHUNK
IDX_ROWS = CHUNK // 128

BLK_B = 64
RB = BLK_B * 200 // 2


def _gather_body(x_hbm, table_hbm, out_hbm, idx_v, rows_v, gsem0, gsem1):
    wid = lax.axis_index("s") * NC + lax.axis_index("c")
    idx_row0 = wid * (PER_W // 128)
    out_row0 = wid * PER_W
    gsems = (gsem0, gsem1)

    def load_idx(ci, b):
        pltpu.sync_copy(
            x_hbm.at[pl.ds(idx_row0 + ci * IDX_ROWS, IDX_ROWS)], idx_v.at[b])

    def fire(b):
        for j in range(IDX_ROWS):
            pltpu.async_copy(table_hbm.at[idx_v.at[b, j]],
                             rows_v.at[b, pl.ds(j * 128, 128)], gsems[b])

    def wait_gathers(b):
        for j in range(IDX_ROWS):
            pltpu.make_async_copy(table_hbm.at[idx_v.at[b, j]],
                                  rows_v.at[b, pl.ds(j * 128, 128)],
                                  gsems[b]).wait()

    def copy_out(ci, b):
        pltpu.sync_copy(rows_v.at[b],
                        out_hbm.at[pl.ds(out_row0 + ci * CHUNK, CHUNK)])

    def step(ci, b):
        nb = 1 - b
        load_idx(ci + 1, nb)
        fire(nb)
        wait_gathers(b)
        copy_out(ci, b)

    load_idx(0, 0)
    fire(0)

    def pair_body(k, carry):
        step(2 * k, 0)
        step(2 * k + 1, 1)
        return carry

    lax.fori_loop(0, N_CHUNKS // 2 - 1, pair_body, 0)
    step(N_CHUNKS - 2, 0)
    wait_gathers(1)
    copy_out(N_CHUNKS - 1, 1)


def _sc_gather(xf, table):
    mesh = plsc.VectorSubcoreMesh(core_axis_name="c", subcore_axis_name="s",
                                  num_cores=NC, num_subcores=NS)
    return pl.kernel(
        _gather_body,
        out_type=jax.ShapeDtypeStruct((BS, EMBED_DIM), jnp.float32),
        mesh=mesh,
        compiler_params=pltpu.CompilerParams(needs_layout_passes=False,
                                             use_tc_tiling_on_sc=False),
        scratch_types=[
            pltpu.VMEM((2, IDX_ROWS, 128), jnp.int32),
            pltpu.VMEM((2, CHUNK, EMBED_DIM), jnp.float32),
            pltpu.SemaphoreType.DMA,
            pltpu.SemaphoreType.DMA,
        ],
    )(xf, table)


def _ln_body(g_ref, gamma_ref, beta_ref, out_ref):
    x = g_ref[...]
    g = gamma_ref[0, :]
    b = beta_ref[0, :]
    gb = jnp.concatenate([g, g])
    bb = jnp.concatenate([b, b])
    lane = lax.broadcasted_iota(jnp.int32, (128, 2), 0)
    half = lax.broadcasted_iota(jnp.int32, (128, 2), 1)
    sel = (lane // EMBED_DIM == half).astype(jnp.float32)
    lane_t = lax.broadcasted_iota(jnp.int32, (2, 128), 1)
    half_t = lax.broadcasted_iota(jnp.int32, (2, 128), 0)
    sel_t = (lane_t // EMBED_DIM == half_t).astype(jnp.float32)

    def mm(a, c):
        return lax.dot_general(a, c, (((1,), (0,)), ((), ())),
                               preferred_element_type=jnp.float32)

    s = mm(x, sel)
    q = mm(x * x, sel)
    mean = s * (1.0 / EMBED_DIM)
    var = q * (1.0 / EMBED_DIM) - mean * mean
    rstd = lax.rsqrt(var + EPS)
    meanb = mm(mean, sel_t)
    rstdb = mm(rstd, sel_t)
    o = (x - meanb) * rstdb * gb + bb
    out_ref[...] = o.reshape(BLK_B, 100, 128)


def _tc_layernorm(g2, gamma2, beta2, s, prev):
    blk0 = s * (BATCH_S // BLK_B)
    args = [g2, gamma2, beta2]
    in_specs = [
        pl.BlockSpec((RB, 128), lambda i: (i, 0)),
        pl.BlockSpec((1, EMBED_DIM), lambda i: (0, 0)),
        pl.BlockSpec((1, EMBED_DIM), lambda i: (0, 0)),
    ]
    aliases = {}
    if prev is not None:
        args.append(prev)
        in_specs.append(pl.BlockSpec(memory_space=pl.ANY))
        aliases = {3: 0}

    def body(*refs):
        _ln_body(refs[0], refs[1], refs[2], refs[-1])

    return pl.pallas_call(
        body,
        grid=(BATCH_S // BLK_B,),
        in_specs=in_specs,
        out_specs=pl.BlockSpec((BLK_B, 100, 128),
                               lambda i: (blk0 + i, 0, 0)),
        out_shape=jax.ShapeDtypeStruct((4096, 100, 128), jnp.float32),
        input_output_aliases=aliases,
    )(*args)


@jax.jit
def kernel(x, table, gamma, beta):
    xi = x.astype(jnp.int32)
    g1 = gamma.reshape(1, EMBED_DIM)
    b1 = beta.reshape(1, EMBED_DIM)
    out = None
    for s in range(S):
        xs = xi[s * BATCH_S:(s + 1) * BATCH_S].reshape(BS // 128, 128)
        g2 = _sc_gather(xs, table).reshape(BS // 2, 128)
        out = _tc_layernorm(g2, g1, b1, s, out)
    return out.reshape(4096, 200, EMBED_DIM)


# speedup vs baseline: 1.5716x; 1.0058x over previous
import jax
import jax.numpy as jnp
from jax import lax
from jax.experimental import pallas as pl
from jax.experimental.pallas import tpu as pltpu
from jax.experimental.pallas import tpu_sc as plsc

NUM_EMBEDDINGS = 1000000
EMBED_DIM = 64
EPS = 1e-5

NC = 2
NS = 16
NW = NC * NS

B = 4096 * 200
S = 4
BATCH_S = 4096 // S
BS = B // S
PER_W = BS // NW
CHUNK = 640
N_CHUNKS = PER_W // CHUNK
IDX_ROWS = CHUNK // 128

BLK_B = 128
RB = BLK_B * 200 // 2


def _gather_body(x_hbm, table_hbm, out_hbm, idx_v, rows_v, gsem0, gsem1):
    wid = lax.axis_index("s") * NC + lax.axis_index("c")
    idx_row0 = wid * (PER_W // 128)
    out_row0 = wid * PER_W
    gsems = (gsem0, gsem1)

    def load_idx(ci, b):
        pltpu.sync_copy(
            x_hbm.at[pl.ds(idx_row0 + ci * IDX_ROWS, IDX_ROWS)], idx_v.at[b])

    def fire(b):
        for j in range(IDX_ROWS):
            pltpu.async_copy(table_hbm.at[idx_v.at[b, j]],
                             rows_v.at[b, pl.ds(j * 128, 128)], gsems[b])

    def wait_gathers(b):
        for j in range(IDX_ROWS):
            pltpu.make_async_copy(table_hbm.at[idx_v.at[b, j]],
                                  rows_v.at[b, pl.ds(j * 128, 128)],
                                  gsems[b]).wait()

    def copy_out(ci, b):
        pltpu.sync_copy(rows_v.at[b],
                        out_hbm.at[pl.ds(out_row0 + ci * CHUNK, CHUNK)])

    def step(ci, b):
        nb = 1 - b
        load_idx(ci + 1, nb)
        fire(nb)
        wait_gathers(b)
        copy_out(ci, b)

    load_idx(0, 0)
    fire(0)

    def pair_body(k, carry):
        step(2 * k, 0)
        step(2 * k + 1, 1)
        return carry

    lax.fori_loop(0, N_CHUNKS // 2 - 1, pair_body, 0)
    step(N_CHUNKS - 2, 0)
    wait_gathers(1)
    copy_out(N_CHUNKS - 1, 1)


def _sc_gather(xf, table):
    mesh = plsc.VectorSubcoreMesh(core_axis_name="c", subcore_axis_name="s",
                                  num_cores=NC, num_subcores=NS)
    return pl.kernel(
        _gather_body,
        out_type=jax.ShapeDtypeStruct((BS, EMBED_DIM), jnp.float32),
        mesh=mesh,
        compiler_params=pltpu.CompilerParams(needs_layout_passes=False,
                                             use_tc_tiling_on_sc=False),
        scratch_types=[
            pltpu.VMEM((2, IDX_ROWS, 128), jnp.int32),
            pltpu.VMEM((2, CHUNK, EMBED_DIM), jnp.float32),
            pltpu.SemaphoreType.DMA,
            pltpu.SemaphoreType.DMA,
        ],
    )(xf, table)


def _ln_body(g_ref, gamma_ref, beta_ref, out_ref):
    x = g_ref[...]
    g = gamma_ref[0, :]
    b = beta_ref[0, :]
    gb = jnp.concatenate([g, g])
    bb = jnp.concatenate([b, b])
    lane = lax.broadcasted_iota(jnp.int32, (128, 2), 0)
    half = lax.broadcasted_iota(jnp.int32, (128, 2), 1)
    sel = (lane // EMBED_DIM == half).astype(jnp.float32)
    lane_t = lax.broadcasted_iota(jnp.int32, (2, 128), 1)
    half_t = lax.broadcasted_iota(jnp.int32, (2, 128), 0)
    sel_t = (lane_t // EMBED_DIM == half_t).astype(jnp.float32)

    def mm(a, c):
        return lax.dot_general(a, c, (((1,), (0,)), ((), ())),
                               preferred_element_type=jnp.float32)

    s = mm(x, sel)
    q = mm(x * x, sel)
    mean = s * (1.0 / EMBED_DIM)
    var = q * (1.0 / EMBED_DIM) - mean * mean
    rstd = lax.rsqrt(var + EPS)
    meanb = mm(mean, sel_t)
    rstdb = mm(rstd, sel_t)
    o = (x - meanb) * rstdb * gb + bb
    out_ref[...] = o.reshape(BLK_B, 100, 128)


def _tc_layernorm(g2, gamma2, beta2, s, prev):
    blk0 = s * (BATCH_S // BLK_B)
    args = [g2, gamma2, beta2]
    in_specs = [
        pl.BlockSpec((RB, 128), lambda i: (i, 0)),
        pl.BlockSpec((1, EMBED_DIM), lambda i: (0, 0)),
        pl.BlockSpec((1, EMBED_DIM), lambda i: (0, 0)),
    ]
    aliases = {}
    if prev is not None:
        args.append(prev)
        in_specs.append(pl.BlockSpec(memory_space=pl.ANY))
        aliases = {3: 0}

    def body(*refs):
        _ln_body(refs[0], refs[1], refs[2], refs[-1])

    return pl.pallas_call(
        body,
        grid=(BATCH_S // BLK_B,),
        in_specs=in_specs,
        out_specs=pl.BlockSpec((BLK_B, 100, 128),
                               lambda i: (blk0 + i, 0, 0)),
        out_shape=jax.ShapeDtypeStruct((4096, 100, 128), jnp.float32),
        input_output_aliases=aliases,
    )(*args)


@jax.jit
def kernel(x, table, gamma, beta):
    xi = x.astype(jnp.int32)
    g1 = gamma.reshape(1, EMBED_DIM)
    b1 = beta.reshape(1, EMBED_DIM)
    out = None
    for s in range(S):
        xs = xi[s * BATCH_S:(s + 1) * BATCH_S].reshape(BS // 128, 128)
        g2 = _sc_gather(xs, table).reshape(BS // 2, 128)
        out = _tc_layernorm(g2, g1, b1, s, out)
    return out.reshape(4096, 200, EMBED_DIM)
